# baseline (speedup 1.0000x reference)
import jax
import jax.numpy as jnp
from jax.experimental import pallas as pl

NUM_NEG_CENTROIDS = 16
NUM_NEG_FEATURES = 128

NPAD = 100352


def _sim_body(z_ref, fb_ref, out_ref):
    eps = 1e-10
    fb = fb_ref[...]
    nrm = jnp.sqrt(jnp.sum(fb * fb, axis=1, keepdims=True)) + eps
    fbn = fb / nrm
    j = pl.program_id(0)
    sim = jax.lax.dot_general(
        z_ref[...], fbn, (((1,), (1,)), ((), ())),
        preferred_element_type=jnp.float32,
        precision=jax.lax.Precision.DEFAULT)
    col = j * 1024 + jax.lax.broadcasted_iota(jnp.int32, sim.shape, 1)
    out_ref[...] = jnp.where(col < 100000, sim, -jnp.inf)


def _nodiag_cols(n):
    j = jnp.arange(n - 1)[None, :]
    i = jnp.arange(n)[:, None]
    return jnp.where(j >= i, j + 1, j)


def kernel(z, feature_bank, centroids, label_bank, idx):
    eps = 1e-10
    zn = z / (jnp.linalg.norm(z, axis=1, keepdims=True) + eps)
    two_n = zn.shape[0]
    n = two_n // 2
    s = zn @ zn.T
    nd_cols = _nodiag_cols(two_n)
    s_nd = jnp.take_along_axis(s, nd_cols, axis=1)
    rows = jnp.arange(two_n)
    pos_cols = jnp.repeat(2 * jnp.arange(n), 2)
    ins_pos = s_nd[rows, pos_cols][:, None]
    j = jnp.arange(two_n - 2)[None, :]
    neg_cols = jnp.where(j >= pos_cols[:, None], j + 1, j)
    ins_neg = jnp.take_along_axis(s_nd, neg_cols, axis=1)
    sq = jnp.sum(centroids ** 2, axis=1)
    d2 = sq[:, None] + sq[None, :] - 2.0 * (centroids @ centroids.T)
    dis = jnp.sqrt(jnp.maximum(d2, 0.0))
    order = jnp.argsort(dis, axis=1)
    close_clusters = order[:, 1:NUM_NEG_CENTROIDS + 1]
    cls_labels = jnp.take(label_bank, idx, axis=0)

    fb_pad = jnp.pad(feature_bank, ((0, NPAD - feature_bank.shape[0]), (0, 0)))
    sim = pl.pallas_call(
        _sim_body,
        grid=(NPAD // 1024,),
        in_specs=[
            pl.BlockSpec((1024, 256), lambda j: (0, 0)),
            pl.BlockSpec((1024, 256), lambda j: (j, 0)),
        ],
        out_specs=pl.BlockSpec((1024, 1024), lambda j: (0, j)),
        out_shape=jax.ShapeDtypeStruct((1024, NPAD), jnp.float32),
    )(zn, fb_pad)
    topk_vals, topk_idx = jax.lax.top_k(sim, NUM_NEG_FEATURES)
    return (ins_pos, ins_neg, close_clusters, cls_labels, topk_vals, topk_idx)


# speedup vs baseline: 17.1929x; 17.1929x over previous
import functools

import jax
import jax.numpy as jnp
from jax import lax
from jax.experimental import pallas as pl
from jax.experimental.pallas import tpu as pltpu
from jax.experimental.pallas import tpu_sc as plsc

NUM_NEG_CENTROIDS = 16
NUM_NEG_FEATURES = 128

NB = 100000
NPAD = 100352
NT = NPAD // 1024
NG = NPAD // 32
PADG0 = NB // 32
CAPG = 256
CAP = 256
BITER = 22
NEG = -3.0e38
IDXPAD = 0x40000000

NC = 2
NS = 16
NW = NC * NS
RPW = 1024 // NW
LPW = 512 // NW


def _sim_body(zn_ref, fb_ref, sel_ref, sim_ref, m32_ref):
    j = pl.program_id(0)
    sim = lax.dot_general(
        zn_ref[...], fb_ref[...], (((1,), (1,)), ((), ())),
        preferred_element_type=jnp.float32, precision=lax.Precision.DEFAULT)
    col = j * 1024 + lax.broadcasted_iota(jnp.int32, (1024, 1024), 1)
    sim = jnp.where(col < NB, sim, NEG)
    sim_ref[...] = sim
    m = sim
    for sh in (16, 8, 4, 2, 1):
        m = jnp.maximum(m, pltpu.roll(m, 1024 - sh, 1))
    m32_ref[...] = lax.dot_general(
        m, sel_ref[...], (((1,), (0,)), ((), ())),
        preferred_element_type=jnp.float32,
        precision=lax.Precision.HIGHEST).reshape(1, 1024, 32)


def _tau_body(m32_ref, tau_ref):
    m32 = m32_ref[...]

    def it(_, c):
        lo, hi = c
        mid = 0.5 * (lo + hi)
        cnt = jnp.sum((m32 >= mid).astype(jnp.float32), axis=1, keepdims=True)
        ge = cnt >= 128.0
        return jnp.where(ge, mid, lo), jnp.where(ge, hi, mid)

    lo = jnp.full((1024, 1), -1.1, jnp.float32)
    hi = jnp.full((1024, 1), 1.1, jnp.float32)
    lo, hi = lax.fori_loop(0, BITER, it, (lo, hi))
    tau_ref[...] = lo


def _sc_body(simc, m32, tau, lab, bidx,
             vout, iout, lout,
             m32_v, gid_v, pid_v, gath_v, val_v, idx_v, tau_v, lidx_v, lab_v,
             sem, sem2):
    cid = lax.axis_index("c")
    sid = lax.axis_index("s")
    wid = sid * NC + cid
    lanes = lax.broadcasted_iota(jnp.int32, (16,), 0)

    pltpu.sync_copy(bidx.at[pl.ds(wid * LPW, LPW)], lidx_v)
    pltpu.async_copy(lab.at[lidx_v], lab_v, sem2).wait()
    pltpu.sync_copy(lab_v, lout.at[pl.ds(wid * LPW, LPW)])

    pltpu.sync_copy(tau.at[pl.ds(wid * RPW, RPW)], tau_v)

    def row_body(k, _):
        r = wid * RPW + k
        base = r * NG
        base128 = r * (NPAD // 128)
        tau_s = plsc.load_gather(tau_v, [jnp.full((16,), k, jnp.int32)])
        pltpu.sync_copy(m32.at[r], m32_v)

        for t in range(CAPG // 16):
            pad_g = PADG0 + ((t * 16 + lanes) & 7)
            gid_v[t // 8, pl.ds((t % 8) * 16, 16)] = base + pad_g
            pid_v[t // 8, pl.ds((t % 8) * 16, 16)] = base128 + (pad_g >> 2)

        def scan_body(i, cnt):
            v = m32_v[pl.ds(i * 16, 16)]
            msk = v >= tau_s
            mi = msk.astype(jnp.int32)
            pos = cnt + plsc.cumsum(mi) - 1
            ok = msk & (pos < CAPG)
            g = i * 16 + lanes
            plsc.store_scatter(gid_v, [pos >> 7, pos & 127], base + g,
                               mask=ok)
            plsc.store_scatter(pid_v, [pos >> 7, pos & 127],
                               base128 + (g >> 2), mask=ok)
            return cnt + jnp.sum(mi)

        cnt = lax.fori_loop(0, NG // 16, scan_body, 0)
        gcnt = jnp.minimum(cnt, CAPG)

        cp0 = pltpu.async_copy(simc.at[pid_v.at[0]], gath_v.at[0], sem)
        cp1 = pltpu.async_copy(simc.at[pid_v.at[1]], gath_v.at[1], sem)
        cp0.wait()
        cp1.wait()

        for t in range(CAP // 16):
            val_v[pl.ds(t * 16, 16)] = jnp.full((16,), NEG, jnp.float32)
            idx_v[pl.ds(t * 16, 16)] = IDXPAD + t * 16 + lanes

        def filt_body(p, scnt):
            prow = jnp.full((16,), p >> 7, jnp.int32)
            pcol = jnp.full((16,), p & 127, jnp.int32)
            gval = plsc.load_gather(gid_v, [prow, pcol])
            g_local = gval - base
            sub = (gval & 3) * 32
            for h in range(2):
                vv = plsc.load_gather(gath_v,
                                      [prow, pcol, sub + h * 16 + lanes])
                msk = vv >= tau_s
                mi = msk.astype(jnp.int32)
                pos = scnt + plsc.cumsum(mi) - 1
                ok = msk & (pos < CAP)
                cols = g_local * 32 + h * 16 + lanes
                plsc.store_scatter(val_v, [pos], vv, mask=ok)
                plsc.store_scatter(idx_v, [pos], cols, mask=ok)
                scnt = scnt + jnp.sum(mi)
            return scnt

        lax.fori_loop(0, gcnt, filt_body, 0)

        pltpu.sync_copy(val_v, vout.at[r])
        pltpu.sync_copy(idx_v, iout.at[r])
        return 0

    lax.fori_loop(0, RPW, row_body, 0)


def _sort_body(v_ref, i_ref, tv_ref, ti_ref):
    v = v_ref[...]
    ix = i_ref[...]
    c = lax.broadcasted_iota(jnp.int32, (1024, CAP), 1)
    k = 2
    while k <= CAP:
        j = k // 2
        while j >= 1:
            vp = jnp.where((c & j) == 0,
                           pltpu.roll(v, CAP - j, 1), pltpu.roll(v, j, 1))
            ip = jnp.where((c & j) == 0,
                           pltpu.roll(ix, CAP - j, 1), pltpu.roll(ix, j, 1))
            better = (v > vp) | ((v == vp) & (ix < ip))
            want = ((c & k) == 0) == ((c & j) == 0)
            take_self = better == want
            v = jnp.where(take_self, v, vp)
            ix = jnp.where(take_self, ix, ip)
            j //= 2
        k *= 2
    tv_ref[...] = v[:, :NUM_NEG_FEATURES]
    ti_ref[...] = ix[:, :NUM_NEG_FEATURES]


def _ins_body(zn_ref, pos_ref, neg_ref):
    zn = zn_ref[...]
    s = lax.dot_general(
        zn, zn, (((1,), (1,)), ((), ())),
        preferred_element_type=jnp.float32, precision=lax.Precision.DEFAULT)
    row = lax.broadcasted_iota(jnp.int32, (1024, 1024), 0)
    col = lax.broadcasted_iota(jnp.int32, (1024, 1024), 1)
    partner = jnp.where(col == (row ^ 1), s, 0.0)
    pos_ref[...] = jnp.sum(partner, axis=1, keepdims=True)
    a = s[:, 0:1022]
    b = s[:, 2:1024]
    row2 = lax.broadcasted_iota(jnp.int32, (1024, 1022), 0)
    col2 = lax.broadcasted_iota(jnp.int32, (1024, 1022), 1)
    neg_ref[...] = jnp.where(col2 < (row2 & ~1), a, b)


def _cent_body(c_ref, sqc_ref, sqr_ref, cc_ref):
    cc = lax.dot_general(
        c_ref[...], c_ref[...], (((1,), (1,)), ((), ())),
        preferred_element_type=jnp.float32, precision=lax.Precision.DEFAULT)
    col = lax.broadcasted_iota(jnp.int32, (1024, 1024), 1)
    d2 = (sqc_ref[...] + sqr_ref[...]) - 2.0 * cc
    dis = jnp.sqrt(jnp.maximum(d2, 0.0))
    dis = jnp.where(col < 1000, dis, 3.0e38)

    def it(t, carry):
        dis, out = carry
        m = jnp.min(dis, axis=1, keepdims=True)
        sel = jnp.min(jnp.where(dis == m, col, 0x7FFFFFFF), axis=1,
                      keepdims=True)
        out = jnp.where(
            lax.broadcasted_iota(jnp.int32, (1024, 32), 1) == (t - 1),
            sel, out)
        dis = jnp.where(col == sel, 3.0e38, dis)
        return dis, out

    out0 = jnp.zeros((1024, 32), jnp.int32)
    _, out = lax.fori_loop(0, NUM_NEG_CENTROIDS + 1, it, (dis, out0))
    cc_ref[...] = out


def kernel(z, feature_bank, centroids, label_bank, idx):
    eps = 1e-10
    zn = z / (jnp.linalg.norm(z, axis=1, keepdims=True) + eps)
    bank_n = feature_bank / (jnp.linalg.norm(feature_bank, axis=1,
                                             keepdims=True) + eps)
    bank_pad = jnp.pad(bank_n, ((0, NPAD - NB), (0, 0)))
    sel = (jnp.arange(1024)[:, None] == 32 * jnp.arange(32)[None, :]).astype(
        jnp.float32)

    sim, m32 = pl.pallas_call(
        _sim_body,
        grid=(NT,),
        in_specs=[
            pl.BlockSpec((1024, 256), lambda j: (0, 0)),
            pl.BlockSpec((1024, 256), lambda j: (j, 0)),
            pl.BlockSpec((1024, 32), lambda j: (0, 0)),
        ],
        out_specs=[
            pl.BlockSpec((1024, 1024), lambda j: (0, j)),
            pl.BlockSpec((1, 1024, 32), lambda j: (j, 0, 0)),
        ],
        out_shape=[
            jax.ShapeDtypeStruct((1024, NPAD), jnp.float32),
            jax.ShapeDtypeStruct((NT, 1024, 32), jnp.float32),
        ],
    )(zn, bank_pad, sel)
    m32 = jnp.transpose(m32, (1, 0, 2)).reshape(1024, NG)

    tau = pl.pallas_call(
        _tau_body,
        out_shape=jax.ShapeDtypeStruct((1024, 1), jnp.float32),
    )(m32)

    simc = sim.reshape(1024 * (NPAD // 128), 128)
    sc_fn = pl.kernel(
        _sc_body,
        out_type=(
            jax.ShapeDtypeStruct((1024, CAP), jnp.float32),
            jax.ShapeDtypeStruct((1024, CAP), jnp.int32),
            jax.ShapeDtypeStruct((512,), jnp.int32),
        ),
        mesh=plsc.VectorSubcoreMesh(core_axis_name="c", subcore_axis_name="s"),
        compiler_params=pltpu.CompilerParams(needs_layout_passes=False),
        scratch_types=[
            pltpu.VMEM((NG,), jnp.float32),
            pltpu.VMEM((2, 128), jnp.int32),
            pltpu.VMEM((2, 128), jnp.int32),
            pltpu.VMEM((2, 128, 128), jnp.float32),
            pltpu.VMEM((CAP,), jnp.float32),
            pltpu.VMEM((CAP,), jnp.int32),
            pltpu.VMEM((RPW,), jnp.float32),
            pltpu.VMEM((LPW,), jnp.int32),
            pltpu.VMEM((LPW,), jnp.int32),
            pltpu.SemaphoreType.DMA,
            pltpu.SemaphoreType.DMA,
        ],
    )
    cand_v, cand_i, cls_labels = sc_fn(
        simc, m32, tau.reshape(1024), label_bank, idx)

    topk_vals, topk_idx = pl.pallas_call(
        _sort_body,
        out_shape=[
            jax.ShapeDtypeStruct((1024, NUM_NEG_FEATURES), jnp.float32),
            jax.ShapeDtypeStruct((1024, NUM_NEG_FEATURES), jnp.int32),
        ],
    )(cand_v, cand_i)

    ins_pos, ins_neg = pl.pallas_call(
        _ins_body,
        out_shape=[
            jax.ShapeDtypeStruct((1024, 1), jnp.float32),
            jax.ShapeDtypeStruct((1024, 1022), jnp.float32),
        ],
    )(zn)

    cpad = jnp.pad(centroids, ((0, 24), (0, 0)))
    sq = jnp.sum(centroids ** 2, axis=1)
    sqpad = jnp.pad(sq, (0, 24))
    cc_out = pl.pallas_call(
        _cent_body,
        out_shape=jax.ShapeDtypeStruct((1024, 32), jnp.int32),
    )(cpad, sqpad.reshape(1024, 1), sqpad.reshape(1, 1024))
    close_clusters = cc_out[:1000, :NUM_NEG_CENTROIDS]

    return (ins_pos, ins_neg, close_clusters, cls_labels, topk_vals, topk_idx)


# speedup vs baseline: 19.8203x; 1.1528x over previous
import functools

import jax
import jax.numpy as jnp
from jax import lax
from jax.experimental import pallas as pl
from jax.experimental.pallas import tpu as pltpu
from jax.experimental.pallas import tpu_sc as plsc

NUM_NEG_CENTROIDS = 16
NUM_NEG_FEATURES = 128

NB = 100000
NPAD = 100352
NT = NPAD // 1024
NG = NPAD // 32
PADG0 = NB // 32
CAPG = 256
CAP = 256
BITER = 22
NEG = -3.0e38
IDXPAD = 0x40000000

NC = 2
NS = 16
NW = NC * NS
RPW = 1024 // NW
LPW = 512 // NW


def _sim_body(zn_ref, fb_ref, sel_ref, sim_ref, m32_ref):
    j = pl.program_id(0)
    sim = lax.dot_general(
        zn_ref[...], fb_ref[...], (((1,), (1,)), ((), ())),
        preferred_element_type=jnp.float32, precision=lax.Precision.DEFAULT)
    col = j * 1024 + lax.broadcasted_iota(jnp.int32, (1024, 1024), 1)
    sim = jnp.where(col < NB, sim, NEG)
    sim_ref[...] = sim.reshape(1024, 1, 8, 128)
    m = sim
    for sh in (16, 8, 4, 2, 1):
        m = jnp.maximum(m, pltpu.roll(m, 1024 - sh, 1))
    m32_ref[...] = lax.dot_general(
        m, sel_ref[...], (((1,), (0,)), ((), ())),
        preferred_element_type=jnp.float32,
        precision=lax.Precision.HIGHEST).reshape(1, 1024, 32)


def _tau_body(m32_ref, tau_ref):
    m32 = m32_ref[...]

    def it(_, c):
        lo, hi = c
        mid = 0.5 * (lo + hi)
        cnt = jnp.sum((m32 >= mid).astype(jnp.float32), axis=1, keepdims=True)
        ge = cnt >= 128.0
        return jnp.where(ge, mid, lo), jnp.where(ge, hi, mid)

    lo = jnp.full((1024, 1), -1.1, jnp.float32)
    hi = jnp.full((1024, 1), 1.1, jnp.float32)
    lo, hi = lax.fori_loop(0, BITER, it, (lo, hi))
    tau_ref[...] = lo


def _sc_body(simc, m32, tau, lab, bidx,
             vout, iout, lout,
             m32_v, gid_v, pid_v, gath_v, val_v, idx_v, tau_v, lidx_v, lab_v,
             sem, sem2):
    cid = lax.axis_index("c")
    sid = lax.axis_index("s")
    wid = sid * NC + cid
    lanes = lax.broadcasted_iota(jnp.int32, (16,), 0)

    pltpu.sync_copy(bidx.at[pl.ds(wid * LPW, LPW)], lidx_v)
    pltpu.async_copy(lab.at[lidx_v], lab_v, sem2).wait()
    pltpu.sync_copy(lab_v, lout.at[pl.ds(wid * LPW, LPW)])

    pltpu.sync_copy(tau.at[pl.ds(wid * RPW, RPW)], tau_v)

    def row_body(k, _):
        r = wid * RPW + k
        base = r * NG
        base128 = r * (NPAD // 128)
        tau_s = plsc.load_gather(tau_v, [jnp.full((16,), k, jnp.int32)])
        pltpu.sync_copy(m32.at[r], m32_v)

        for t in range(CAPG // 16):
            pad_g = PADG0 + ((t * 16 + lanes) & 7)
            gid_v[t // 8, pl.ds((t % 8) * 16, 16)] = base + pad_g
            pid_v[t // 8, pl.ds((t % 8) * 16, 16)] = base128 + (pad_g >> 2)

        def scan_body(i, cnt):
            v = m32_v[pl.ds(i * 16, 16)]
            msk = v >= tau_s
            mi = msk.astype(jnp.int32)
            pos = cnt + plsc.cumsum(mi) - 1
            ok = msk & (pos < CAPG)
            g = i * 16 + lanes
            plsc.store_scatter(gid_v, [pos >> 7, pos & 127], base + g,
                               mask=ok)
            plsc.store_scatter(pid_v, [pos >> 7, pos & 127],
                               base128 + (g >> 2), mask=ok)
            return cnt + jnp.sum(mi)

        cnt = lax.fori_loop(0, NG // 16, scan_body, 0)
        gcnt = jnp.minimum(cnt, CAPG)

        cp0 = pltpu.async_copy(simc.at[pid_v.at[0]], gath_v.at[0], sem)

        @pl.when(cnt > 128)
        def _():
            pltpu.async_copy(simc.at[pid_v.at[1]], gath_v.at[1], sem).wait()

        cp0.wait()

        for t in range(CAP // 16):
            val_v[pl.ds(t * 16, 16)] = jnp.full((16,), NEG, jnp.float32)
            idx_v[pl.ds(t * 16, 16)] = IDXPAD + t * 16 + lanes

        def filt_body(p, scnt):
            prow = jnp.full((16,), p >> 7, jnp.int32)
            pcol = jnp.full((16,), p & 127, jnp.int32)
            gval = plsc.load_gather(gid_v, [prow, pcol])
            g_local = gval - base
            sub = (gval & 3) * 32
            for h in range(2):
                vv = plsc.load_gather(gath_v,
                                      [prow, pcol, sub + h * 16 + lanes])
                msk = vv >= tau_s
                mi = msk.astype(jnp.int32)
                pos = scnt + plsc.cumsum(mi) - 1
                ok = msk & (pos < CAP)
                cols = g_local * 32 + h * 16 + lanes
                plsc.store_scatter(val_v, [pos], vv, mask=ok)
                plsc.store_scatter(idx_v, [pos], cols, mask=ok)
                scnt = scnt + jnp.sum(mi)
            return scnt

        lax.fori_loop(0, gcnt, filt_body, 0)

        pltpu.sync_copy(val_v, vout.at[r])
        pltpu.sync_copy(idx_v, iout.at[r])
        return 0

    lax.fori_loop(0, RPW, row_body, 0)


def _sort_body(v_ref, i_ref, tv_ref, ti_ref):
    v = v_ref[...]
    ix = i_ref[...]
    c = lax.broadcasted_iota(jnp.int32, (1024, CAP), 1)
    k = 2
    while k <= CAP:
        j = k // 2
        while j >= 1:
            vp = jnp.where((c & j) == 0,
                           pltpu.roll(v, CAP - j, 1), pltpu.roll(v, j, 1))
            ip = jnp.where((c & j) == 0,
                           pltpu.roll(ix, CAP - j, 1), pltpu.roll(ix, j, 1))
            better = (v > vp) | ((v == vp) & (ix < ip))
            want = ((c & k) == 0) == ((c & j) == 0)
            take_self = better == want
            v = jnp.where(take_self, v, vp)
            ix = jnp.where(take_self, ix, ip)
            j //= 2
        k *= 2
    tv_ref[...] = v[:, :NUM_NEG_FEATURES]
    ti_ref[...] = ix[:, :NUM_NEG_FEATURES]


def _ins_body(zn_ref, pos_ref, neg_ref):
    zn = zn_ref[...]
    s = lax.dot_general(
        zn, zn, (((1,), (1,)), ((), ())),
        preferred_element_type=jnp.float32, precision=lax.Precision.DEFAULT)
    row = lax.broadcasted_iota(jnp.int32, (1024, 1024), 0)
    col = lax.broadcasted_iota(jnp.int32, (1024, 1024), 1)
    partner = jnp.where(col == (row ^ 1), s, 0.0)
    pos_ref[...] = jnp.sum(partner, axis=1, keepdims=True)
    a = s[:, 0:1022]
    b = s[:, 2:1024]
    row2 = lax.broadcasted_iota(jnp.int32, (1024, 1022), 0)
    col2 = lax.broadcasted_iota(jnp.int32, (1024, 1022), 1)
    neg_ref[...] = jnp.where(col2 < (row2 & ~1), a, b)


def _cent_body(c_ref, sqc_ref, sqr_ref, cc_ref):
    cc = lax.dot_general(
        c_ref[...], c_ref[...], (((1,), (1,)), ((), ())),
        preferred_element_type=jnp.float32, precision=lax.Precision.DEFAULT)
    col = lax.broadcasted_iota(jnp.int32, (1024, 1024), 1)
    d2 = (sqc_ref[...] + sqr_ref[...]) - 2.0 * cc
    dis = jnp.sqrt(jnp.maximum(d2, 0.0))
    dis = jnp.where(col < 1000, dis, 3.0e38)

    def it(t, carry):
        dis, out = carry
        m = jnp.min(dis, axis=1, keepdims=True)
        sel = jnp.min(jnp.where(dis == m, col, 0x7FFFFFFF), axis=1,
                      keepdims=True)
        out = jnp.where(
            lax.broadcasted_iota(jnp.int32, (1024, 32), 1) == (t - 1),
            sel, out)
        dis = jnp.where(col == sel, 3.0e38, dis)
        return dis, out

    out0 = jnp.zeros((1024, 32), jnp.int32)
    _, out = lax.fori_loop(0, NUM_NEG_CENTROIDS + 1, it, (dis, out0))
    cc_ref[...] = out


def kernel(z, feature_bank, centroids, label_bank, idx):
    eps = 1e-10
    zn = z / (jnp.linalg.norm(z, axis=1, keepdims=True) + eps)
    bank_n = feature_bank / (jnp.linalg.norm(feature_bank, axis=1,
                                             keepdims=True) + eps)
    bank_pad = jnp.pad(bank_n, ((0, NPAD - NB), (0, 0)))
    sel = (jnp.arange(1024)[:, None] == 32 * jnp.arange(32)[None, :]).astype(
        jnp.float32)

    sim, m32 = pl.pallas_call(
        _sim_body,
        grid=(NT,),
        in_specs=[
            pl.BlockSpec((1024, 256), lambda j: (0, 0)),
            pl.BlockSpec((1024, 256), lambda j: (j, 0)),
            pl.BlockSpec((1024, 32), lambda j: (0, 0)),
        ],
        out_specs=[
            pl.BlockSpec((1024, 1, 8, 128), lambda j: (0, j, 0, 0)),
            pl.BlockSpec((1, 1024, 32), lambda j: (j, 0, 0)),
        ],
        out_shape=[
            jax.ShapeDtypeStruct((1024, NT, 8, 128), jnp.float32),
            jax.ShapeDtypeStruct((NT, 1024, 32), jnp.float32),
        ],
    )(zn, bank_pad, sel)
    m32 = jnp.transpose(m32, (1, 0, 2)).reshape(1024, NG)

    tau = pl.pallas_call(
        _tau_body,
        out_shape=jax.ShapeDtypeStruct((1024, 1), jnp.float32),
    )(m32)

    simc = sim.reshape(1024 * (NPAD // 128), 128)
    sc_fn = pl.kernel(
        _sc_body,
        out_type=(
            jax.ShapeDtypeStruct((1024, CAP), jnp.float32),
            jax.ShapeDtypeStruct((1024, CAP), jnp.int32),
            jax.ShapeDtypeStruct((512,), jnp.int32),
        ),
        mesh=plsc.VectorSubcoreMesh(core_axis_name="c", subcore_axis_name="s"),
        compiler_params=pltpu.CompilerParams(needs_layout_passes=False),
        scratch_types=[
            pltpu.VMEM((NG,), jnp.float32),
            pltpu.VMEM((2, 128), jnp.int32),
            pltpu.VMEM((2, 128), jnp.int32),
            pltpu.VMEM((2, 128, 128), jnp.float32),
            pltpu.VMEM((CAP,), jnp.float32),
            pltpu.VMEM((CAP,), jnp.int32),
            pltpu.VMEM((RPW,), jnp.float32),
            pltpu.VMEM((LPW,), jnp.int32),
            pltpu.VMEM((LPW,), jnp.int32),
            pltpu.SemaphoreType.DMA,
            pltpu.SemaphoreType.DMA,
        ],
    )
    cand_v, cand_i, cls_labels = sc_fn(
        simc, m32, tau.reshape(1024), label_bank, idx)

    topk_vals, topk_idx = pl.pallas_call(
        _sort_body,
        out_shape=[
            jax.ShapeDtypeStruct((1024, NUM_NEG_FEATURES), jnp.float32),
            jax.ShapeDtypeStruct((1024, NUM_NEG_FEATURES), jnp.int32),
        ],
    )(cand_v, cand_i)

    ins_pos, ins_neg = pl.pallas_call(
        _ins_body,
        out_shape=[
            jax.ShapeDtypeStruct((1024, 1), jnp.float32),
            jax.ShapeDtypeStruct((1024, 1022), jnp.float32),
        ],
    )(zn)

    cpad = jnp.pad(centroids, ((0, 24), (0, 0)))
    sq = jnp.sum(centroids ** 2, axis=1)
    sqpad = jnp.pad(sq, (0, 24))
    cc_out = pl.pallas_call(
        _cent_body,
        out_shape=jax.ShapeDtypeStruct((1024, 32), jnp.int32),
    )(cpad, sqpad.reshape(1024, 1), sqpad.reshape(1, 1024))
    close_clusters = cc_out[:1000, :NUM_NEG_CENTROIDS]

    return (ins_pos, ins_neg, close_clusters, cls_labels, topk_vals, topk_idx)


# speedup vs baseline: 20.9653x; 1.0578x over previous
import functools

import jax
import jax.numpy as jnp
from jax import lax
from jax.experimental import pallas as pl
from jax.experimental.pallas import tpu as pltpu
from jax.experimental.pallas import tpu_sc as plsc

NUM_NEG_CENTROIDS = 16
NUM_NEG_FEATURES = 128

NB = 100000
NPAD = 100352
NT = NPAD // 1024
NG = NPAD // 32
PADG0 = NB // 32
CAPG = 256
CAP = 256
BITER = 22
NEG = -3.0e38
IDXPAD = 0x40000000

NC = 2
NS = 16
NW = NC * NS
RPW = 1024 // NW
LPW = 512 // NW


def _sim_body(zn_ref, fb_ref, nrm_ref, sel_ref, sim_ref, m32_ref):
    j = pl.program_id(0)
    fbn = fb_ref[...] / (nrm_ref[...] + 1e-10)
    sim = lax.dot_general(
        zn_ref[...], fbn, (((1,), (1,)), ((), ())),
        preferred_element_type=jnp.float32, precision=lax.Precision.DEFAULT)
    col = j * 1024 + lax.broadcasted_iota(jnp.int32, (1024, 1024), 1)
    sim = jnp.where(col < NB, sim, NEG)
    sim_ref[...] = sim.reshape(1024, 1, 8, 128)
    m = sim
    for sh in (16, 8, 4, 2, 1):
        m = jnp.maximum(m, pltpu.roll(m, 1024 - sh, 1))
    m32_ref[...] = lax.dot_general(
        m, sel_ref[...], (((1,), (0,)), ((), ())),
        preferred_element_type=jnp.float32,
        precision=lax.Precision.HIGHEST).reshape(1, 1024, 32)


def _tau_body(m32_ref, tau_ref):
    m32 = m32_ref[...]

    def it(_, c):
        lo, hi = c
        mid = 0.5 * (lo + hi)
        cnt = jnp.sum((m32 >= mid).astype(jnp.float32), axis=1, keepdims=True)
        ge = cnt >= 128.0
        return jnp.where(ge, mid, lo), jnp.where(ge, hi, mid)

    lo = jnp.full((1024, 1), -1.1, jnp.float32)
    hi = jnp.full((1024, 1), 1.1, jnp.float32)
    lo, hi = lax.fori_loop(0, BITER, it, (lo, hi))
    tau_ref[...] = lo


def _sc_body(simc, m32, tau, lab, bidx,
             vout, iout, lout,
             m32_v, gid_v, pid_v, gath_v, val_v, idx_v, tau_v, lidx_v, lab_v,
             sem, sem2):
    cid = lax.axis_index("c")
    sid = lax.axis_index("s")
    wid = sid * NC + cid
    lanes = lax.broadcasted_iota(jnp.int32, (16,), 0)

    pltpu.sync_copy(bidx.at[pl.ds(wid * LPW, LPW)], lidx_v)
    pltpu.async_copy(lab.at[lidx_v], lab_v, sem2).wait()
    pltpu.sync_copy(lab_v, lout.at[pl.ds(wid * LPW, LPW)])

    pltpu.sync_copy(tau.at[pl.ds(wid * RPW, RPW)], tau_v)

    def row_body(k, _):
        r = wid * RPW + k
        base = r * NG
        base128 = r * (NPAD // 128)
        tau_s = plsc.load_gather(tau_v, [jnp.full((16,), k, jnp.int32)])
        pltpu.sync_copy(m32.at[r], m32_v)

        for t in range(CAPG // 16):
            pad_g = PADG0 + ((t * 16 + lanes) & 7)
            gid_v[t // 8, pl.ds((t % 8) * 16, 16)] = base + pad_g
            pid_v[t // 8, pl.ds((t % 8) * 16, 16)] = base128 + (pad_g >> 2)

        def scan_body(i, cnt):
            v = m32_v[pl.ds(i * 16, 16)]
            msk = v >= tau_s
            mi = msk.astype(jnp.int32)
            pos = cnt + plsc.cumsum(mi) - 1
            ok = msk & (pos < CAPG)
            g = i * 16 + lanes
            plsc.store_scatter(gid_v, [pos >> 7, pos & 127], base + g,
                               mask=ok)
            plsc.store_scatter(pid_v, [pos >> 7, pos & 127],
                               base128 + (g >> 2), mask=ok)
            return cnt + jnp.sum(mi)

        cnt = lax.fori_loop(0, NG // 16, scan_body, 0)
        gcnt = jnp.minimum(cnt, CAPG)

        cp0 = pltpu.async_copy(simc.at[pid_v.at[0]], gath_v.at[0], sem)

        @pl.when(cnt > 128)
        def _():
            pltpu.async_copy(simc.at[pid_v.at[1]], gath_v.at[1], sem).wait()

        cp0.wait()

        for t in range(CAP // 16):
            val_v[pl.ds(t * 16, 16)] = jnp.full((16,), NEG, jnp.float32)
            idx_v[pl.ds(t * 16, 16)] = IDXPAD + t * 16 + lanes

        def filt_body(p, scnt):
            prow = jnp.full((16,), p >> 7, jnp.int32)
            pcol = jnp.full((16,), p & 127, jnp.int32)
            gval = plsc.load_gather(gid_v, [prow, pcol])
            g_local = gval - base
            sub = (gval & 3) * 32
            for h in range(2):
                vv = plsc.load_gather(gath_v,
                                      [prow, pcol, sub + h * 16 + lanes])
                msk = vv >= tau_s
                mi = msk.astype(jnp.int32)
                pos = scnt + plsc.cumsum(mi) - 1
                ok = msk & (pos < CAP)
                cols = g_local * 32 + h * 16 + lanes
                plsc.store_scatter(val_v, [pos], vv, mask=ok)
                plsc.store_scatter(idx_v, [pos], cols, mask=ok)
                scnt = scnt + jnp.sum(mi)
            return scnt

        lax.fori_loop(0, gcnt, filt_body, 0)

        pltpu.sync_copy(val_v, vout.at[r])
        pltpu.sync_copy(idx_v, iout.at[r])
        return 0

    lax.fori_loop(0, RPW, row_body, 0)


def _sort_body(v_ref, i_ref, tv_ref, ti_ref):
    v = v_ref[...]
    ix = i_ref[...]
    c = lax.broadcasted_iota(jnp.int32, (1024, CAP), 1)
    k = 2
    while k <= CAP:
        j = k // 2
        while j >= 1:
            vp = jnp.where((c & j) == 0,
                           pltpu.roll(v, CAP - j, 1), pltpu.roll(v, j, 1))
            ip = jnp.where((c & j) == 0,
                           pltpu.roll(ix, CAP - j, 1), pltpu.roll(ix, j, 1))
            better = (v > vp) | ((v == vp) & (ix < ip))
            want = ((c & k) == 0) == ((c & j) == 0)
            take_self = better == want
            v = jnp.where(take_self, v, vp)
            ix = jnp.where(take_self, ix, ip)
            j //= 2
        k *= 2
    tv_ref[...] = v[:, :NUM_NEG_FEATURES]
    ti_ref[...] = ix[:, :NUM_NEG_FEATURES]


def _ins_body(zn_ref, pos_ref, neg_ref):
    zn = zn_ref[...]
    s = lax.dot_general(
        zn, zn, (((1,), (1,)), ((), ())),
        preferred_element_type=jnp.float32, precision=lax.Precision.DEFAULT)
    row = lax.broadcasted_iota(jnp.int32, (1024, 1024), 0)
    col = lax.broadcasted_iota(jnp.int32, (1024, 1024), 1)
    partner = jnp.where(col == (row ^ 1), s, 0.0)
    pos_ref[...] = jnp.sum(partner, axis=1, keepdims=True)
    a = s[:, 0:1022]
    b = s[:, 2:1024]
    row2 = lax.broadcasted_iota(jnp.int32, (1024, 1022), 0)
    col2 = lax.broadcasted_iota(jnp.int32, (1024, 1022), 1)
    neg_ref[...] = jnp.where(col2 < (row2 & ~1), a, b)


def _cent_body(c_ref, sqc_ref, sqr_ref, cc_ref):
    cc = lax.dot_general(
        c_ref[...], c_ref[...], (((1,), (1,)), ((), ())),
        preferred_element_type=jnp.float32, precision=lax.Precision.DEFAULT)
    col = lax.broadcasted_iota(jnp.int32, (1024, 1024), 1)
    d2 = (sqc_ref[...] + sqr_ref[...]) - 2.0 * cc
    dis = jnp.sqrt(jnp.maximum(d2, 0.0))
    dis = jnp.where(col < 1000, dis, 3.0e38)

    def it(t, carry):
        dis, out = carry
        m = jnp.min(dis, axis=1, keepdims=True)
        sel = jnp.min(jnp.where(dis == m, col, 0x7FFFFFFF), axis=1,
                      keepdims=True)
        out = jnp.where(
            lax.broadcasted_iota(jnp.int32, (1024, 32), 1) == (t - 1),
            sel, out)
        dis = jnp.where(col == sel, 3.0e38, dis)
        return dis, out

    out0 = jnp.zeros((1024, 32), jnp.int32)
    _, out = lax.fori_loop(0, NUM_NEG_CENTROIDS + 1, it, (dis, out0))
    cc_ref[...] = out


def kernel(z, feature_bank, centroids, label_bank, idx):
    eps = 1e-10
    zn = z / (jnp.linalg.norm(z, axis=1, keepdims=True) + eps)
    nrm = jnp.linalg.norm(feature_bank, axis=1, keepdims=True)
    sel = (jnp.arange(1024)[:, None] == 32 * jnp.arange(32)[None, :]).astype(
        jnp.float32)

    sim, m32 = pl.pallas_call(
        _sim_body,
        grid=(NT,),
        in_specs=[
            pl.BlockSpec((1024, 256), lambda j: (0, 0)),
            pl.BlockSpec((1024, 256), lambda j: (j, 0)),
            pl.BlockSpec((1024, 1), lambda j: (j, 0)),
            pl.BlockSpec((1024, 32), lambda j: (0, 0)),
        ],
        out_specs=[
            pl.BlockSpec((1024, 1, 8, 128), lambda j: (0, j, 0, 0)),
            pl.BlockSpec((1, 1024, 32), lambda j: (j, 0, 0)),
        ],
        out_shape=[
            jax.ShapeDtypeStruct((1024, NT, 8, 128), jnp.float32),
            jax.ShapeDtypeStruct((NT, 1024, 32), jnp.float32),
        ],
    )(zn, feature_bank, nrm, sel)
    m32 = jnp.transpose(m32, (1, 0, 2)).reshape(1024, NG)

    tau = pl.pallas_call(
        _tau_body,
        out_shape=jax.ShapeDtypeStruct((1024, 1), jnp.float32),
    )(m32)

    simc = sim.reshape(1024 * (NPAD // 128), 128)
    sc_fn = pl.kernel(
        _sc_body,
        out_type=(
            jax.ShapeDtypeStruct((1024, CAP), jnp.float32),
            jax.ShapeDtypeStruct((1024, CAP), jnp.int32),
            jax.ShapeDtypeStruct((512,), jnp.int32),
        ),
        mesh=plsc.VectorSubcoreMesh(core_axis_name="c", subcore_axis_name="s"),
        compiler_params=pltpu.CompilerParams(needs_layout_passes=False),
        scratch_types=[
            pltpu.VMEM((NG,), jnp.float32),
            pltpu.VMEM((2, 128), jnp.int32),
            pltpu.VMEM((2, 128), jnp.int32),
            pltpu.VMEM((2, 128, 128), jnp.float32),
            pltpu.VMEM((CAP,), jnp.float32),
            pltpu.VMEM((CAP,), jnp.int32),
            pltpu.VMEM((RPW,), jnp.float32),
            pltpu.VMEM((LPW,), jnp.int32),
            pltpu.VMEM((LPW,), jnp.int32),
            pltpu.SemaphoreType.DMA,
            pltpu.SemaphoreType.DMA,
        ],
    )
    cand_v, cand_i, cls_labels = sc_fn(
        simc, m32, tau.reshape(1024), label_bank, idx)

    topk_vals, topk_idx = pl.pallas_call(
        _sort_body,
        out_shape=[
            jax.ShapeDtypeStruct((1024, NUM_NEG_FEATURES), jnp.float32),
            jax.ShapeDtypeStruct((1024, NUM_NEG_FEATURES), jnp.int32),
        ],
    )(cand_v, cand_i)

    ins_pos, ins_neg = pl.pallas_call(
        _ins_body,
        out_shape=[
            jax.ShapeDtypeStruct((1024, 1), jnp.float32),
            jax.ShapeDtypeStruct((1024, 1022), jnp.float32),
        ],
    )(zn)

    cpad = jnp.pad(centroids, ((0, 24), (0, 0)))
    sq = jnp.sum(centroids ** 2, axis=1)
    sqpad = jnp.pad(sq, (0, 24))
    cc_out = pl.pallas_call(
        _cent_body,
        out_shape=jax.ShapeDtypeStruct((1024, 32), jnp.int32),
    )(cpad, sqpad.reshape(1024, 1), sqpad.reshape(1, 1024))
    close_clusters = cc_out[:1000, :NUM_NEG_CENTROIDS]

    return (ins_pos, ins_neg, close_clusters, cls_labels, topk_vals, topk_idx)


# speedup vs baseline: 21.1520x; 1.0089x over previous
import functools

import jax
import jax.numpy as jnp
from jax import lax
from jax.experimental import pallas as pl
from jax.experimental.pallas import tpu as pltpu
from jax.experimental.pallas import tpu_sc as plsc

NUM_NEG_CENTROIDS = 16
NUM_NEG_FEATURES = 128

NB = 100000
NPAD = 100352
NT = NPAD // 1024
NG = NPAD // 32
PADG0 = NB // 32
CAPG = 256
CAP = 256
BITER = 22
NEG = -3.0e38
IDXPAD = 0x40000000

NC = 2
NS = 16
NW = NC * NS
RPW = 1024 // NW
LPW = 512 // NW


def _sim_body(zn_ref, fb_ref, nrm_ref, sel_ref, sim_ref, m32_ref):
    j = pl.program_id(0)
    fbn = fb_ref[...] / (nrm_ref[...] + 1e-10)
    sim = lax.dot_general(
        zn_ref[...], fbn, (((1,), (1,)), ((), ())),
        preferred_element_type=jnp.float32, precision=lax.Precision.DEFAULT)
    col = j * 1024 + lax.broadcasted_iota(jnp.int32, (1024, 1024), 1)
    sim = jnp.where(col < NB, sim, NEG)
    sim_ref[...] = sim.reshape(1024, 1, 8, 128)
    m = sim
    for sh in (16, 8, 4, 2, 1):
        m = jnp.maximum(m, pltpu.roll(m, 1024 - sh, 1))
    m32_ref[...] = lax.dot_general(
        m, sel_ref[...], (((1,), (0,)), ((), ())),
        preferred_element_type=jnp.float32,
        precision=lax.Precision.HIGHEST).reshape(1, 1024, 32)


def _tau_body(m32_ref, tau_ref):
    m32 = m32_ref[...]

    def it(_, c):
        lo, hi = c
        mid = 0.5 * (lo + hi)
        cnt = jnp.sum((m32 >= mid).astype(jnp.float32), axis=1, keepdims=True)
        ge = cnt >= 128.0
        return jnp.where(ge, mid, lo), jnp.where(ge, hi, mid)

    lo = jnp.full((1024, 1), -1.1, jnp.float32)
    hi = jnp.full((1024, 1), 1.1, jnp.float32)
    lo, hi = lax.fori_loop(0, BITER, it, (lo, hi))
    tau_ref[...] = lo


def _sc_body(simc, m32, tau, lab, bidx,
             vout, iout, lout,
             m32_v, gid_v, pid_v, gath_v, val_v, idx_v, tau_v, lidx_v, lab_v,
             sem, sem2, sem3):
    cid = lax.axis_index("c")
    sid = lax.axis_index("s")
    wid = sid * NC + cid
    lanes = lax.broadcasted_iota(jnp.int32, (16,), 0)

    pltpu.sync_copy(bidx.at[pl.ds(wid * LPW, LPW)], lidx_v)
    pltpu.async_copy(lab.at[lidx_v], lab_v, sem2).wait()
    pltpu.sync_copy(lab_v, lout.at[pl.ds(wid * LPW, LPW)])

    pltpu.sync_copy(tau.at[pl.ds(wid * RPW, RPW)], tau_v)

    pltpu.async_copy(m32.at[wid * RPW], m32_v.at[0], sem3)

    def row_body(k, _):
        r = wid * RPW + k
        base = r * NG
        base128 = r * (NPAD // 128)
        cur = k & 1
        tau_s = plsc.load_gather(tau_v, [jnp.full((16,), k, jnp.int32)])
        pltpu.make_async_copy(m32.at[r], m32_v.at[cur], sem3).wait()

        @pl.when(k + 1 < RPW)
        def _():
            pltpu.async_copy(m32.at[r + 1], m32_v.at[(k + 1) & 1],
                             sem3)

        for t in range(CAPG // 16):
            pad_g = PADG0 + ((t * 16 + lanes) & 7)
            gid_v[t // 8, pl.ds((t % 8) * 16, 16)] = base + pad_g
            pid_v[t // 8, pl.ds((t % 8) * 16, 16)] = base128 + (pad_g >> 2)

        def scan_body(i, cnt):
            v = m32_v[cur, pl.ds(i * 16, 16)]
            msk = v >= tau_s
            mi = msk.astype(jnp.int32)
            pos = cnt + plsc.cumsum(mi) - 1
            ok = msk & (pos < CAPG)
            g = i * 16 + lanes
            plsc.store_scatter(gid_v, [pos >> 7, pos & 127], base + g,
                               mask=ok)
            plsc.store_scatter(pid_v, [pos >> 7, pos & 127],
                               base128 + (g >> 2), mask=ok)
            return cnt + jnp.sum(mi)

        cnt = lax.fori_loop(0, NG // 16, scan_body, 0)
        gcnt = jnp.minimum(cnt, CAPG)

        cp0 = pltpu.async_copy(simc.at[pid_v.at[0]], gath_v.at[0], sem)

        @pl.when(cnt > 128)
        def _():
            pltpu.async_copy(simc.at[pid_v.at[1]], gath_v.at[1], sem)

        for t in range(CAP // 16):
            val_v[pl.ds(t * 16, 16)] = jnp.full((16,), NEG, jnp.float32)
            idx_v[pl.ds(t * 16, 16)] = IDXPAD + t * 16 + lanes

        cp0.wait()

        @pl.when(cnt > 128)
        def _():
            pltpu.make_async_copy(
                simc.at[pid_v.at[1]], gath_v.at[1], sem).wait()

        def filt_body(p, scnt):
            prow = jnp.full((16,), p >> 7, jnp.int32)
            pcol = jnp.full((16,), p & 127, jnp.int32)
            gval = plsc.load_gather(gid_v, [prow, pcol])
            g_local = gval - base
            sub = (gval & 3) * 32
            for h in range(2):
                vv = plsc.load_gather(gath_v,
                                      [prow, pcol, sub + h * 16 + lanes])
                msk = vv >= tau_s
                mi = msk.astype(jnp.int32)
                pos = scnt + plsc.cumsum(mi) - 1
                ok = msk & (pos < CAP)
                cols = g_local * 32 + h * 16 + lanes
                plsc.store_scatter(val_v, [pos], vv, mask=ok)
                plsc.store_scatter(idx_v, [pos], cols, mask=ok)
                scnt = scnt + jnp.sum(mi)
            return scnt

        lax.fori_loop(0, gcnt, filt_body, 0)

        pltpu.sync_copy(val_v, vout.at[r])
        pltpu.sync_copy(idx_v, iout.at[r])
        return 0

    lax.fori_loop(0, RPW, row_body, 0)


def _sort_body(v_ref, i_ref, tv_ref, ti_ref):
    v = v_ref[...]
    ix = i_ref[...]
    c = lax.broadcasted_iota(jnp.int32, (1024, CAP), 1)
    k = 2
    while k <= CAP:
        j = k // 2
        while j >= 1:
            vp = jnp.where((c & j) == 0,
                           pltpu.roll(v, CAP - j, 1), pltpu.roll(v, j, 1))
            ip = jnp.where((c & j) == 0,
                           pltpu.roll(ix, CAP - j, 1), pltpu.roll(ix, j, 1))
            better = (v > vp) | ((v == vp) & (ix < ip))
            want = ((c & k) == 0) == ((c & j) == 0)
            take_self = better == want
            v = jnp.where(take_self, v, vp)
            ix = jnp.where(take_self, ix, ip)
            j //= 2
        k *= 2
    tv_ref[...] = v[:, :NUM_NEG_FEATURES]
    ti_ref[...] = ix[:, :NUM_NEG_FEATURES]


def _ins_body(zn_ref, pos_ref, neg_ref):
    zn = zn_ref[...]
    s = lax.dot_general(
        zn, zn, (((1,), (1,)), ((), ())),
        preferred_element_type=jnp.float32, precision=lax.Precision.DEFAULT)
    row = lax.broadcasted_iota(jnp.int32, (1024, 1024), 0)
    col = lax.broadcasted_iota(jnp.int32, (1024, 1024), 1)
    partner = jnp.where(col == (row ^ 1), s, 0.0)
    pos_ref[...] = jnp.sum(partner, axis=1, keepdims=True)
    a = s[:, 0:1022]
    b = s[:, 2:1024]
    row2 = lax.broadcasted_iota(jnp.int32, (1024, 1022), 0)
    col2 = lax.broadcasted_iota(jnp.int32, (1024, 1022), 1)
    neg_ref[...] = jnp.where(col2 < (row2 & ~1), a, b)


def _cent_body(c_ref, sqc_ref, sqr_ref, cc_ref):
    cc = lax.dot_general(
        c_ref[...], c_ref[...], (((1,), (1,)), ((), ())),
        preferred_element_type=jnp.float32, precision=lax.Precision.DEFAULT)
    col = lax.broadcasted_iota(jnp.int32, (1024, 1024), 1)
    d2 = (sqc_ref[...] + sqr_ref[...]) - 2.0 * cc
    dis = jnp.sqrt(jnp.maximum(d2, 0.0))
    dis = jnp.where(col < 1000, dis, 3.0e38)

    def it(t, carry):
        dis, out = carry
        m = jnp.min(dis, axis=1, keepdims=True)
        sel = jnp.min(jnp.where(dis == m, col, 0x7FFFFFFF), axis=1,
                      keepdims=True)
        out = jnp.where(
            lax.broadcasted_iota(jnp.int32, (1024, 32), 1) == (t - 1),
            sel, out)
        dis = jnp.where(col == sel, 3.0e38, dis)
        return dis, out

    out0 = jnp.zeros((1024, 32), jnp.int32)
    _, out = lax.fori_loop(0, NUM_NEG_CENTROIDS + 1, it, (dis, out0))
    cc_ref[...] = out


def kernel(z, feature_bank, centroids, label_bank, idx):
    eps = 1e-10
    zn = z / (jnp.linalg.norm(z, axis=1, keepdims=True) + eps)
    nrm = jnp.linalg.norm(feature_bank, axis=1, keepdims=True)
    sel = (jnp.arange(1024)[:, None] == 32 * jnp.arange(32)[None, :]).astype(
        jnp.float32)

    sim, m32 = pl.pallas_call(
        _sim_body,
        grid=(NT,),
        in_specs=[
            pl.BlockSpec((1024, 256), lambda j: (0, 0)),
            pl.BlockSpec((1024, 256), lambda j: (j, 0)),
            pl.BlockSpec((1024, 1), lambda j: (j, 0)),
            pl.BlockSpec((1024, 32), lambda j: (0, 0)),
        ],
        out_specs=[
            pl.BlockSpec((1024, 1, 8, 128), lambda j: (0, j, 0, 0)),
            pl.BlockSpec((1, 1024, 32), lambda j: (j, 0, 0)),
        ],
        out_shape=[
            jax.ShapeDtypeStruct((1024, NT, 8, 128), jnp.float32),
            jax.ShapeDtypeStruct((NT, 1024, 32), jnp.float32),
        ],
    )(zn, feature_bank, nrm, sel)
    m32 = jnp.transpose(m32, (1, 0, 2)).reshape(1024, NG)

    tau = pl.pallas_call(
        _tau_body,
        out_shape=jax.ShapeDtypeStruct((1024, 1), jnp.float32),
    )(m32)

    simc = sim.reshape(1024 * (NPAD // 128), 128)
    sc_fn = pl.kernel(
        _sc_body,
        out_type=(
            jax.ShapeDtypeStruct((1024, CAP), jnp.float32),
            jax.ShapeDtypeStruct((1024, CAP), jnp.int32),
            jax.ShapeDtypeStruct((512,), jnp.int32),
        ),
        mesh=plsc.VectorSubcoreMesh(core_axis_name="c", subcore_axis_name="s"),
        compiler_params=pltpu.CompilerParams(needs_layout_passes=False),
        scratch_types=[
            pltpu.VMEM((2, NG), jnp.float32),
            pltpu.VMEM((2, 128), jnp.int32),
            pltpu.VMEM((2, 128), jnp.int32),
            pltpu.VMEM((2, 128, 128), jnp.float32),
            pltpu.VMEM((CAP,), jnp.float32),
            pltpu.VMEM((CAP,), jnp.int32),
            pltpu.VMEM((RPW,), jnp.float32),
            pltpu.VMEM((LPW,), jnp.int32),
            pltpu.VMEM((LPW,), jnp.int32),
            pltpu.SemaphoreType.DMA,
            pltpu.SemaphoreType.DMA,
            pltpu.SemaphoreType.DMA,
        ],
    )
    cand_v, cand_i, cls_labels = sc_fn(
        simc, m32, tau.reshape(1024), label_bank, idx)

    topk_vals, topk_idx = pl.pallas_call(
        _sort_body,
        out_shape=[
            jax.ShapeDtypeStruct((1024, NUM_NEG_FEATURES), jnp.float32),
            jax.ShapeDtypeStruct((1024, NUM_NEG_FEATURES), jnp.int32),
        ],
    )(cand_v, cand_i)

    ins_pos, ins_neg = pl.pallas_call(
        _ins_body,
        out_shape=[
            jax.ShapeDtypeStruct((1024, 1), jnp.float32),
            jax.ShapeDtypeStruct((1024, 1022), jnp.float32),
        ],
    )(zn)

    cpad = jnp.pad(centroids, ((0, 24), (0, 0)))
    sq = jnp.sum(centroids ** 2, axis=1)
    sqpad = jnp.pad(sq, (0, 24))
    cc_out = pl.pallas_call(
        _cent_body,
        out_shape=jax.ShapeDtypeStruct((1024, 32), jnp.int32),
    )(cpad, sqpad.reshape(1024, 1), sqpad.reshape(1, 1024))
    close_clusters = cc_out[:1000, :NUM_NEG_CENTROIDS]

    return (ins_pos, ins_neg, close_clusters, cls_labels, topk_vals, topk_idx)
